# baseline (speedup 1.0000x reference)
import functools

import jax
import jax.numpy as jnp
import numpy as np
from jax import lax
from jax.experimental import pallas as pl
from jax.experimental.pallas import tpu as pltpu
from jax.experimental.pallas import tpu_sc as plsc

_B, _S, _D = 16, 4096, 128
_N = _B * _S
_E = _N * 5
_K = int(_E * 0.3)
_RB = 1024
_NBLK = _N // _RB
_EB = _RB * 5


def _tf_core(k1, k2, x1, x2):
    rot = ((13, 15, 26, 6), (17, 29, 16, 24))

    def rotl(x, d):
        return ((x << np.uint32(d)) | (x >> np.uint32(32 - d))).astype(np.uint32)

    ks = [np.uint32(k1), np.uint32(k2),
          np.uint32(np.uint32(k1) ^ np.uint32(k2) ^ np.uint32(0x1BD11BDA))]
    x = [x1.astype(np.uint32) + ks[0], x2.astype(np.uint32) + ks[1]]
    for i in range(5):
        for r in rot[i % 2]:
            x[0] = (x[0] + x[1]).astype(np.uint32)
            x[1] = rotl(x[1], r) ^ x[0]
        x[0] = (x[0] + ks[(i + 1) % 3]).astype(np.uint32)
        x[1] = (x[1] + ks[(i + 2) % 3] + np.uint32(i + 1)).astype(np.uint32)
    return x[0], x[1]


def _np_randint(seed, shape, minval, maxval):
    o1, o2 = _tf_core(0, seed, np.zeros(2, np.uint32),
                      np.arange(2, dtype=np.uint32))
    keys = np.stack([o1, o2], axis=1)
    size = int(np.prod(shape))

    def bits(key):
        b1, b2 = _tf_core(key[0], key[1], np.zeros(size, np.uint32),
                          np.arange(size, dtype=np.uint32))
        return b1 ^ b2

    hi, lo = bits(keys[0]), bits(keys[1])
    span = np.uint32(maxval - minval)
    mult = np.uint32((int(2 ** 16 % span) * int(2 ** 16 % span)) % int(span))
    off = ((hi % span) * mult + (lo % span)) % span
    return (minval + off.astype(np.int64)).astype(np.int32).reshape(shape)


def _topology():
    rnd = _np_randint(42, (_N, 5), 1, _S)
    dst = ((np.arange(_N, dtype=np.int64)[:, None] + rnd) % _S).astype(np.int32)
    return dst.reshape(-1)


_TOPO = _topology()
_FOLD5 = (np.arange(640)[:, None] // 5 == np.arange(128)[None, :]).astype(np.float32)

_NW = 32
_EW = _E // _NW


def _sc_mesh():
    return plsc.VectorSubcoreMesh(core_axis_name="c", subcore_axis_name="s",
                                  num_cores=2, num_subcores=16)


def _sc_gatherB(Bm, dstarr):
    @functools.partial(
        pl.kernel,
        out_type=jax.ShapeDtypeStruct((_E, 128), jnp.float32),
        mesh=_sc_mesh(),
        scratch_types=[pltpu.VMEM((512,), jnp.int32),
                       pltpu.VMEM((512, 128), jnp.float32),
                       pltpu.SemaphoreType.DMA],
    )
    def k(b_hbm, dst_hbm, g_hbm, idx_v, rows_v, sem):
        wid = lax.axis_index("s") * 2 + lax.axis_index("c")
        base = wid * _EW

        def chunk(ci, carry):
            row0 = base + ci * 512
            pltpu.sync_copy(dst_hbm.at[pl.ds(row0, 512)], idx_v)
            cps = [pltpu.async_copy(b_hbm.at[idx_v.at[pl.ds(kk * 128, 128)]],
                                    rows_v.at[pl.ds(kk * 128, 128), :], sem)
                   for kk in range(4)]
            for cp in cps:
                cp.wait()
            pltpu.sync_copy(rows_v, g_hbm.at[pl.ds(row0, 512)])
            return carry

        lax.fori_loop(0, _EW // 512, chunk, 0)

    return k(Bm, dstarr)


def _sc_deg(m, dstarr):
    @functools.partial(
        pl.kernel,
        out_type=jax.ShapeDtypeStruct((_NW, _S), jnp.float32),
        mesh=_sc_mesh(),
        compiler_params=pltpu.CompilerParams(needs_layout_passes=False),
        scratch_types=[pltpu.VMEM((2048,), jnp.float32),
                       pltpu.VMEM((2048,), jnp.int32),
                       pltpu.VMEM((_S,), jnp.float32)],
    )
    def k(m_hbm, dst_hbm, out_hbm, m_v, dst_v, deg_v):
        wid = lax.axis_index("s") * 2 + lax.axis_index("c")
        base = wid * _EW
        zero16 = jnp.zeros((16,), jnp.float32)

        def z(i, c):
            deg_v[pl.ds(i * 16, 16)] = zero16
            return c

        lax.fori_loop(0, _S // 16, z, 0)

        def chunk(ci, c):
            pltpu.sync_copy(m_hbm.at[pl.ds(base + ci * 2048, 2048)], m_v)
            pltpu.sync_copy(dst_hbm.at[pl.ds(base + ci * 2048, 2048)], dst_v)

            def inner(kk, c2):
                idx = dst_v[pl.ds(kk * 16, 16)]
                vals = m_v[pl.ds(kk * 16, 16)]
                plsc.addupdate_scatter(deg_v, [idx], vals)
                return c2

            lax.fori_loop(0, 128, inner, 0)
            return c

        lax.fori_loop(0, _EW // 2048, chunk, 0)
        pltpu.sync_copy(deg_v, out_hbm.at[wid])

    return k(m, dstarr)


def _k_reduce(degp_ref, m_ref, rep_ref, dinv_ref, dsq_ref, mdrep_ref):
    deg = 1.0 + jnp.sum(degp_ref[...], axis=0)
    dinv = jax.lax.rsqrt(deg)
    dinv_ref[...] = dinv
    dsq_ref[...] = dinv * dinv
    dfull = jnp.concatenate(
        [dinv, jnp.ones((480, 128), jnp.float32)], axis=0)
    dinvrep = jnp.dot(dfull, rep_ref[...],
                      preferred_element_type=jnp.float32)
    md = m_ref[...] * dinvrep
    mdrep_ref[...] = jnp.concatenate(
        [md, jnp.zeros((1, 640), jnp.float32)], axis=0)


def _reduce_deg(degp, m2d):
    return pl.pallas_call(
        _k_reduce,
        in_specs=[pl.BlockSpec((_NW, 32, 128), lambda: (0, 0, 0)),
                  pl.BlockSpec((512, 640), lambda: (0, 0)),
                  pl.BlockSpec((128, 640), lambda: (0, 0))],
        out_specs=[pl.BlockSpec((32, 128), lambda: (0, 0)),
                   pl.BlockSpec((32, 128), lambda: (0, 0)),
                   pl.BlockSpec((513, 640), lambda: (0, 0))],
        out_shape=[jax.ShapeDtypeStruct((32, 128), jnp.float32),
                   jax.ShapeDtypeStruct((32, 128), jnp.float32),
                   jax.ShapeDtypeStruct((513, 640), jnp.float32)],
    )(degp.reshape(_NW, 32, 128), m2d.reshape(512, 640),
      jnp.asarray(_FOLD5.T))


def _k_dense(x_ref, w1_ref, e1t_ref, eb1_ref, g_ref, a_ref):
    xb = x_ref[...]
    g_ref[...] = jnp.dot(xb, w1_ref[...], preferred_element_type=jnp.float32)
    a_ref[...] = jnp.dot(xb, e1t_ref[...],
                         preferred_element_type=jnp.float32) + eb1_ref[...]


def _dense(xf, W1, Eg1t, eb1):
    return pl.pallas_call(
        _k_dense,
        grid=(_NBLK,),
        in_specs=[pl.BlockSpec((_RB, _D), lambda i: (i, 0)),
                  pl.BlockSpec((_D, _D), lambda i: (0, 0)),
                  pl.BlockSpec((_D, 64), lambda i: (0, 0)),
                  pl.BlockSpec((1, 64), lambda i: (0, 0))],
        out_specs=[pl.BlockSpec((_RB, _D), lambda i: (i, 0)),
                   pl.BlockSpec((_RB, 64), lambda i: (i, 0))],
        out_shape=[jax.ShapeDtypeStruct((_N, _D), jnp.float32),
                   jax.ShapeDtypeStruct((_N, 64), jnp.float32)],
    )(xf, W1, Eg1t, eb1.reshape(1, 64))


def _k_small_mm(x_ref, w_ref, o_ref):
    o_ref[...] = jnp.dot(x_ref[...], w_ref[...],
                         preferred_element_type=jnp.float32)


def _bmat(x0, Eg1b):
    return pl.pallas_call(
        _k_small_mm,
        grid=(4,),
        in_specs=[pl.BlockSpec((_RB, _D), lambda i: (i, 0)),
                  pl.BlockSpec((_D, _D), lambda i: (0, 0))],
        out_specs=pl.BlockSpec((_RB, _D), lambda i: (i, 0)),
        out_shape=jax.ShapeDtypeStruct((_S, _D), jnp.float32),
    )(x0, Eg1b)


def _k_score(a_ref, gth_ref, eg2_ref, eb2_ref, o_ref):
    a = a_ref[...]
    ar = jnp.broadcast_to(a[:, None, :], (_RB, 5, 64)).reshape(_EB, 64)
    r = jnp.maximum(ar + gth_ref[...][:, :64], 0.0)
    z = jnp.sum(r * eg2_ref[...], axis=1, keepdims=True) + eb2_ref[...]
    ew = jax.nn.sigmoid(z)
    o_ref[...] = jax.lax.bitcast_convert_type(ew, jnp.int32)


def _score(A, G, Eg2, eb2):
    return pl.pallas_call(
        _k_score,
        grid=(_NBLK,),
        in_specs=[pl.BlockSpec((_RB, 64), lambda i: (i, 0)),
                  pl.BlockSpec((_EB, 128), lambda i: (i, 0)),
                  pl.BlockSpec((1, 64), lambda i: (0, 0)),
                  pl.BlockSpec((1, 1), lambda i: (0, 0))],
        out_specs=pl.BlockSpec((_EB, 1), lambda i: (i, 0)),
        out_shape=jax.ShapeDtypeStruct((_E, 1), jnp.int32),
    )(A, G, Eg2.reshape(1, 64), eb2.reshape(1, 1))


_ROWS = _E // 128


def _k_mask(bits_ref, m_ref):
    bits = bits_ref[...]

    def cnt_ge(v):
        return jnp.sum((bits >= v).astype(jnp.int32))

    def bis_val(_, c):
        lo, hi = c
        mid = lo + (hi - lo + 1) // 2
        ok = cnt_ge(mid) >= _K
        return jnp.where(ok, mid, lo), jnp.where(ok, hi, mid - 1)

    t, _ = jax.lax.fori_loop(0, 31, bis_val,
                             (jnp.int32(0), jnp.int32(0x3F800000)))
    cnt_gt = cnt_ge(t + 1)
    extra = _K - cnt_gt
    tie = bits == t
    idx = (jax.lax.broadcasted_iota(jnp.int32, (_ROWS, 128), 0) * 128
           + jax.lax.broadcasted_iota(jnp.int32, (_ROWS, 128), 1))

    def bis_idx(_, c):
        lo, hi = c
        mid = lo + (hi - lo + 1) // 2
        ok = jnp.sum((tie & (idx <= mid)).astype(jnp.int32)) <= extra
        return jnp.where(ok, mid, lo), jnp.where(ok, hi, mid - 1)

    tau, _ = jax.lax.fori_loop(0, 19, bis_idx,
                               (jnp.int32(-1), jnp.int32(_E - 1)))
    m_ref[...] = ((bits > t) | (tie & (idx <= tau))).astype(jnp.float32)


def _mask(bits2d):
    return pl.pallas_call(
        _k_mask,
        in_specs=[pl.BlockSpec((_ROWS, 128), lambda: (0, 0))],
        out_specs=pl.BlockSpec((_ROWS, 128), lambda: (0, 0)),
        out_shape=jax.ShapeDtypeStruct((_ROWS, 128), jnp.float32),
    )(bits2d)


def _k_final(g_ref, acc_ref, w_ref, dsq_ref, r5_ref, b1_ref, w2_ref, b2_ref,
             fw_ref, fb_ref, pool_ref, fb_out_ref, srow, s0row):
    i = pl.program_id(0)
    bat = i // 4

    @pl.when(i == 0)
    def _init():
        srow[...] = jnp.zeros_like(srow)
        s0row[...] = jnp.zeros_like(s0row)

    flag0 = (i < 4).astype(jnp.float32)
    flagn = (i >= 4).astype(jnp.float32)
    h1 = jnp.maximum(g_ref[...] * flagn + acc_ref[...] * flag0
                     + b1_ref[...], 0.0)
    wsum = jnp.dot(w_ref[...], r5_ref[...],
                   preferred_element_type=jnp.float32)
    coef = wsum + dsq_ref[...] * flag0
    part = jnp.zeros((1, _D), jnp.float32)
    for r in range(8):
        part += jnp.dot(coef[r:r + 1, :], h1[128 * r:128 * (r + 1), :],
                        preferred_element_type=jnp.float32)
    s0row[...] += part

    @pl.when(bat >= 1)
    def _colsum():
        cs = jnp.sum(h1, axis=0, keepdims=True)
        srow[pl.ds(bat, 1), :] += cs

    @pl.when(i == _NBLK - 1)
    def _fin():
        Sm = srow[...]
        Sm = jnp.concatenate([s0row[...], Sm[1:, :]], axis=0)
        pooled = jnp.dot(Sm, w2_ref[...],
                         preferred_element_type=jnp.float32) * (1.0 / _S) \
            + b2_ref[...]
        pool_ref[...] = pooled
        fb_out_ref[...] = jax.nn.sigmoid(
            jnp.dot(pooled, fw_ref[...],
                    preferred_element_type=jnp.float32) + fb_ref[...])


def _final(g, accp, w2d, dsq_nm, b1, W2, b2, Fw, Fb):
    return pl.pallas_call(
        _k_final,
        grid=(_NBLK,),
        in_specs=[pl.BlockSpec((_RB, _D), lambda i: (i, 0)),
                  pl.BlockSpec((_RB, _D), lambda i: (jnp.minimum(i, 3), 0)),
                  pl.BlockSpec((8, 640), lambda i: (i, 0)),
                  pl.BlockSpec((8, 128), lambda i: (jnp.minimum(i, 3), 0)),
                  pl.BlockSpec((640, 128), lambda i: (0, 0)),
                  pl.BlockSpec((1, _D), lambda i: (0, 0)),
                  pl.BlockSpec((_D, _D), lambda i: (0, 0)),
                  pl.BlockSpec((1, _D), lambda i: (0, 0)),
                  pl.BlockSpec((_D, _D), lambda i: (0, 0)),
                  pl.BlockSpec((1, _D), lambda i: (0, 0))],
        out_specs=[pl.BlockSpec((_B, _D), lambda i: (0, 0)),
                   pl.BlockSpec((_B, _D), lambda i: (0, 0))],
        out_shape=[jax.ShapeDtypeStruct((_B, _D), jnp.float32),
                   jax.ShapeDtypeStruct((_B, _D), jnp.float32)],
        scratch_shapes=[pltpu.VMEM((_B, _D), jnp.float32),
                        pltpu.VMEM((1, _D), jnp.float32)],
    )(g, accp, w2d, dsq_nm, jnp.asarray(_FOLD5), b1.reshape(1, _D), W2,
      b2.reshape(1, _D), Fw, Fb.reshape(1, _D))


def kernel(x, W1, b1, W2, b2, Eg1, eb1, Eg2, eb2, Fw, Fb):
    dst = jnp.asarray(_TOPO)
    xf = x.reshape(_N, _D)

    g, A = _dense(xf, W1, Eg1[:_D], eb1)
    Bm = _bmat(xf[:_S], jnp.pad(Eg1[_D:], ((0, 0), (0, 64))))
    G = _sc_gatherB(Bm, dst)
    bits = _score(A, G, Eg2, eb2)
    m2d = _mask(bits.reshape(_ROWS, 128))
    m = m2d.reshape(_E)

    degp = _sc_deg(m, dst)
    dinv_nm, dsq_nm, mdrep_ext = _reduce_deg(degp, m2d)
    dinv0 = dinv_nm.reshape(_S)
    dsq = dsq_nm.reshape(_S)
    dv = jnp.take(dinv0, dst)
    w = mdrep_ext.reshape(-1)[:_E] * dv
    src = jnp.repeat(jnp.arange(_N, dtype=jnp.int32), 5)
    acc = jnp.zeros((_S, _D), jnp.float32).at[dst].add(
        w[:, None] * jnp.take(g, src, axis=0))
    accp = acc + dsq[:, None] * g[:_S]
    return _final(g, accp, w.reshape(512, 640), dsq_nm, b1, W2, b2, Fw, Fb)


# speedup vs baseline: 1.7261x; 1.7261x over previous
import functools

import jax
import jax.numpy as jnp
import numpy as np
from jax import lax
from jax.experimental import pallas as pl
from jax.experimental.pallas import tpu as pltpu
from jax.experimental.pallas import tpu_sc as plsc

_B, _S, _D = 16, 4096, 128
_N = _B * _S
_E = _N * 5
_K = int(_E * 0.3)
_RB = 1024
_NBLK = _N // _RB
_EB = _RB * 5


def _tf_core(k1, k2, x1, x2):
    rot = ((13, 15, 26, 6), (17, 29, 16, 24))

    def rotl(x, d):
        return ((x << np.uint32(d)) | (x >> np.uint32(32 - d))).astype(np.uint32)

    ks = [np.uint32(k1), np.uint32(k2),
          np.uint32(np.uint32(k1) ^ np.uint32(k2) ^ np.uint32(0x1BD11BDA))]
    x = [x1.astype(np.uint32) + ks[0], x2.astype(np.uint32) + ks[1]]
    for i in range(5):
        for r in rot[i % 2]:
            x[0] = (x[0] + x[1]).astype(np.uint32)
            x[1] = rotl(x[1], r) ^ x[0]
        x[0] = (x[0] + ks[(i + 1) % 3]).astype(np.uint32)
        x[1] = (x[1] + ks[(i + 2) % 3] + np.uint32(i + 1)).astype(np.uint32)
    return x[0], x[1]


def _np_randint(seed, shape, minval, maxval):
    o1, o2 = _tf_core(0, seed, np.zeros(2, np.uint32),
                      np.arange(2, dtype=np.uint32))
    keys = np.stack([o1, o2], axis=1)
    size = int(np.prod(shape))

    def bits(key):
        b1, b2 = _tf_core(key[0], key[1], np.zeros(size, np.uint32),
                          np.arange(size, dtype=np.uint32))
        return b1 ^ b2

    hi, lo = bits(keys[0]), bits(keys[1])
    span = np.uint32(maxval - minval)
    mult = np.uint32((int(2 ** 16 % span) * int(2 ** 16 % span)) % int(span))
    off = ((hi % span) * mult + (lo % span)) % span
    return (minval + off.astype(np.int64)).astype(np.int32).reshape(shape)


def _topology():
    rnd = _np_randint(42, (_N, 5), 1, _S)
    dst = ((np.arange(_N, dtype=np.int64)[:, None] + rnd) % _S).astype(np.int32)
    return dst.reshape(-1)


_TOPO = _topology()
_FOLD5 = (np.arange(640)[:, None] // 5 == np.arange(128)[None, :]).astype(np.float32)

_NW = 32
_EW = _E // _NW


def _partition():
    tile = _TOPO >> 7
    order = np.argsort(tile, kind="stable").astype(np.int32)
    cnts = np.bincount(tile, minlength=_NW)
    L = int(np.ceil(cnts.max() / 128) * 128)
    eid = np.full((_NW, L), _E, np.int32)
    off = 0
    for t in range(_NW):
        eid[t, :cnts[t]] = order[off:off + cnts[t]]
        off += cnts[t]
    src = np.where(eid < _E, eid // 5, 0).astype(np.int32)
    dstg = np.where(eid < _E, _TOPO[np.minimum(eid, _E - 1)],
                    (np.arange(_NW, dtype=np.int32) * 128 + 128)[:, None])
    return eid, src, dstg.astype(np.int32), L


_EID2, _SRC2, _DSTG2, _LP = _partition()
_NBM = _E // 32


def _sc_acc(bm, dinv0p, g, eid2, src2, dstg2):
    @functools.partial(
        pl.kernel,
        out_type=jax.ShapeDtypeStruct((_S, _D), jnp.float32),
        mesh=_sc_mesh(),
        compiler_params=pltpu.CompilerParams(needs_layout_passes=False),
        scratch_types=[pltpu.VMEM((_NBM + 16,), jnp.int32),
                       pltpu.VMEM((4112,), jnp.float32),
                       pltpu.VMEM((_LP,), jnp.int32),
                       pltpu.VMEM((_LP,), jnp.int32),
                       pltpu.VMEM((_LP,), jnp.int32),
                       pltpu.VMEM((_LP + 16,), jnp.int32),
                       pltpu.VMEM((_LP + 16,), jnp.int32),
                       pltpu.VMEM((_LP + 16,), jnp.float32),
                       pltpu.VMEM((128, _D), jnp.float32),
                       pltpu.VMEM((129, _D), jnp.float32),
                       pltpu.SemaphoreType.DMA],
    )
    def k(bm_hbm, dinv_hbm, g_hbm, eid_hbm, src_hbm, dstg_hbm, acc_hbm,
          bm_v, dinv_v, eid_v, src_v, dstg_v, csrc_v, crow_v, cw_v,
          grow_v, acc_v, sem):
        wid = lax.axis_index("s") * 2 + lax.axis_index("c")
        base = wid * 128
        iota = lax.iota(jnp.int32, 16)

        pltpu.sync_copy(bm_hbm.at[pl.ds(0, _NBM)], bm_v.at[pl.ds(0, _NBM)])
        pltpu.sync_copy(dinv_hbm, dinv_v)
        pltpu.sync_copy(eid_hbm.at[wid], eid_v)
        pltpu.sync_copy(src_hbm.at[wid], src_v)
        pltpu.sync_copy(dstg_hbm.at[wid], dstg_v)
        pltpu.async_copy(g_hbm.at[pl.ds(base, 128)], grow_v, sem).wait()

        def self_loop(r, c):
            d = plsc.load_gather(dinv_v, [jnp.full((16,), base, jnp.int32) + r])
            d2 = d * d
            for v in range(8):
                gval = plsc.load_gather(
                    grow_v, [jnp.full((16,), r, jnp.int32), v * 16 + iota])
                plsc.store_scatter(
                    acc_v, [jnp.full((16,), r, jnp.int32), v * 16 + iota],
                    gval * d2)
            return c

        lax.fori_loop(0, 128, self_loop, 0)

        def fill(i, c):
            csrc_v[pl.ds(i * 16, 16)] = jnp.zeros((16,), jnp.int32)
            crow_v[pl.ds(i * 16, 16)] = jnp.full((16,), 128, jnp.int32)
            cw_v[pl.ds(i * 16, 16)] = jnp.zeros((16,), jnp.float32)
            return c

        lax.fori_loop(0, (_LP + 16) // 16, fill, 0)

        def groups(gi, kcnt):
            eid = eid_v[pl.ds(gi * 16, 16)]
            wi = ((eid >> 12) << 7) | (eid & 127)
            bit = (eid >> 7) & 31
            word = plsc.load_gather(bm_v, [wi])
            mb = ((word >> bit) & 1).astype(jnp.float32)
            srci = src_v[pl.ds(gi * 16, 16)]
            dstg = dstg_v[pl.ds(gi * 16, 16)]
            dfull = jnp.where(
                srci < _S,
                plsc.load_gather(dinv_v, [jnp.minimum(srci, _S - 1)]), 1.0)
            w = mb * dfull * plsc.load_gather(dinv_v, [dstg])
            keep = w != 0.0
            plsc.store_compressed(csrc_v.at[pl.ds(kcnt, 16)], srci, mask=keep)
            plsc.store_compressed(crow_v.at[pl.ds(kcnt, 16)], dstg - base, mask=keep)
            plsc.store_compressed(cw_v.at[pl.ds(kcnt, 16)], w, mask=keep)
            return kcnt + jnp.sum(keep.astype(jnp.int32))

        kcnt = lax.fori_loop(0, _LP // 16, groups, jnp.int32(0))

        def chunk(c2, c):
            pltpu.async_copy(
                g_hbm.at[csrc_v.at[pl.ds(c2 * 128, 128)]], grow_v, sem).wait()

            def edge(e2, c3):
                sp = jnp.full((16,), c2 * 128, jnp.int32) + e2
                wspl = plsc.load_gather(cw_v, [sp])
                rowspl = plsc.load_gather(crow_v, [sp])
                espl = jnp.full((16,), e2, jnp.int32)
                for v in range(8):
                    gval = plsc.load_gather(grow_v, [espl, v * 16 + iota])
                    plsc.addupdate_scatter(
                        acc_v, [rowspl, v * 16 + iota], gval * wspl)
                return c3

            lax.fori_loop(0, 128, edge, 0)
            return c

        lax.fori_loop(0, (kcnt + 127) // 128, chunk, 0)
        pltpu.sync_copy(acc_v.at[pl.ds(0, 128), :], acc_hbm.at[pl.ds(base, 128)])

    return k(bm, dinv0p, g, eid2, src2, dstg2)


def _sc_mesh():
    return plsc.VectorSubcoreMesh(core_axis_name="c", subcore_axis_name="s",
                                  num_cores=2, num_subcores=16)


def _sc_gatherB(Bm, dstarr):
    @functools.partial(
        pl.kernel,
        out_type=jax.ShapeDtypeStruct((_E, 128), jnp.float32),
        mesh=_sc_mesh(),
        scratch_types=[pltpu.VMEM((512,), jnp.int32),
                       pltpu.VMEM((512, 128), jnp.float32),
                       pltpu.SemaphoreType.DMA],
    )
    def k(b_hbm, dst_hbm, g_hbm, idx_v, rows_v, sem):
        wid = lax.axis_index("s") * 2 + lax.axis_index("c")
        base = wid * _EW

        def chunk(ci, carry):
            row0 = base + ci * 512
            pltpu.sync_copy(dst_hbm.at[pl.ds(row0, 512)], idx_v)
            cps = [pltpu.async_copy(b_hbm.at[idx_v.at[pl.ds(kk * 128, 128)]],
                                    rows_v.at[pl.ds(kk * 128, 128), :], sem)
                   for kk in range(4)]
            for cp in cps:
                cp.wait()
            pltpu.sync_copy(rows_v, g_hbm.at[pl.ds(row0, 512)])
            return carry

        lax.fori_loop(0, _EW // 512, chunk, 0)

    return k(Bm, dstarr)


def _sc_deg(m, dstarr):
    @functools.partial(
        pl.kernel,
        out_type=jax.ShapeDtypeStruct((_NW, _S), jnp.float32),
        mesh=_sc_mesh(),
        compiler_params=pltpu.CompilerParams(needs_layout_passes=False),
        scratch_types=[pltpu.VMEM((2048,), jnp.float32),
                       pltpu.VMEM((2048,), jnp.int32),
                       pltpu.VMEM((_S,), jnp.float32)],
    )
    def k(m_hbm, dst_hbm, out_hbm, m_v, dst_v, deg_v):
        wid = lax.axis_index("s") * 2 + lax.axis_index("c")
        base = wid * _EW
        zero16 = jnp.zeros((16,), jnp.float32)

        def z(i, c):
            deg_v[pl.ds(i * 16, 16)] = zero16
            return c

        lax.fori_loop(0, _S // 16, z, 0)

        def chunk(ci, c):
            pltpu.sync_copy(m_hbm.at[pl.ds(base + ci * 2048, 2048)], m_v)
            pltpu.sync_copy(dst_hbm.at[pl.ds(base + ci * 2048, 2048)], dst_v)

            def inner(kk, c2):
                idx = dst_v[pl.ds(kk * 16, 16)]
                vals = m_v[pl.ds(kk * 16, 16)]
                plsc.addupdate_scatter(deg_v, [idx], vals)
                return c2

            lax.fori_loop(0, 128, inner, 0)
            return c

        lax.fori_loop(0, _EW // 2048, chunk, 0)
        pltpu.sync_copy(deg_v, out_hbm.at[wid])

    return k(m, dstarr)


def _k_reduce(degp_ref, m_ref, rep_ref, dinv_ref, dsq_ref, mdrep_ref):
    deg = 1.0 + jnp.sum(degp_ref[...], axis=0)
    dinv = jax.lax.rsqrt(deg)
    dinv_ref[...] = dinv
    dsq_ref[...] = dinv * dinv
    dfull = jnp.concatenate(
        [dinv, jnp.ones((480, 128), jnp.float32)], axis=0)
    dinvrep = jnp.dot(dfull, rep_ref[...],
                      preferred_element_type=jnp.float32)
    md = m_ref[...] * dinvrep
    mdrep_ref[...] = jnp.concatenate(
        [md, jnp.zeros((1, 640), jnp.float32)], axis=0)


def _reduce_deg(degp, m2d):
    return pl.pallas_call(
        _k_reduce,
        in_specs=[pl.BlockSpec((_NW, 32, 128), lambda: (0, 0, 0)),
                  pl.BlockSpec((512, 640), lambda: (0, 0)),
                  pl.BlockSpec((128, 640), lambda: (0, 0))],
        out_specs=[pl.BlockSpec((32, 128), lambda: (0, 0)),
                   pl.BlockSpec((32, 128), lambda: (0, 0)),
                   pl.BlockSpec((513, 640), lambda: (0, 0))],
        out_shape=[jax.ShapeDtypeStruct((32, 128), jnp.float32),
                   jax.ShapeDtypeStruct((32, 128), jnp.float32),
                   jax.ShapeDtypeStruct((513, 640), jnp.float32)],
    )(degp.reshape(_NW, 32, 128), m2d.reshape(512, 640),
      jnp.asarray(_FOLD5.T))


def _k_dense(x_ref, w1_ref, e1t_ref, eb1_ref, g_ref, a_ref):
    xb = x_ref[...]
    g_ref[...] = jnp.dot(xb, w1_ref[...], preferred_element_type=jnp.float32)
    a_ref[...] = jnp.dot(xb, e1t_ref[...],
                         preferred_element_type=jnp.float32) + eb1_ref[...]


def _dense(xf, W1, Eg1t, eb1):
    return pl.pallas_call(
        _k_dense,
        grid=(_NBLK,),
        in_specs=[pl.BlockSpec((_RB, _D), lambda i: (i, 0)),
                  pl.BlockSpec((_D, _D), lambda i: (0, 0)),
                  pl.BlockSpec((_D, 64), lambda i: (0, 0)),
                  pl.BlockSpec((1, 64), lambda i: (0, 0))],
        out_specs=[pl.BlockSpec((_RB, _D), lambda i: (i, 0)),
                   pl.BlockSpec((_RB, 64), lambda i: (i, 0))],
        out_shape=[jax.ShapeDtypeStruct((_N, _D), jnp.float32),
                   jax.ShapeDtypeStruct((_N, 64), jnp.float32)],
    )(xf, W1, Eg1t, eb1.reshape(1, 64))


def _k_small_mm(x_ref, w_ref, o_ref):
    o_ref[...] = jnp.dot(x_ref[...], w_ref[...],
                         preferred_element_type=jnp.float32)


def _bmat(x0, Eg1b):
    return pl.pallas_call(
        _k_small_mm,
        grid=(4,),
        in_specs=[pl.BlockSpec((_RB, _D), lambda i: (i, 0)),
                  pl.BlockSpec((_D, _D), lambda i: (0, 0))],
        out_specs=pl.BlockSpec((_RB, _D), lambda i: (i, 0)),
        out_shape=jax.ShapeDtypeStruct((_S, _D), jnp.float32),
    )(x0, Eg1b)


def _k_score(a_ref, gth_ref, eg2_ref, eb2_ref, o_ref):
    a = a_ref[...]
    ar = jnp.broadcast_to(a[:, None, :], (_RB, 5, 64)).reshape(_EB, 64)
    r = jnp.maximum(ar + gth_ref[...][:, :64], 0.0)
    z = jnp.sum(r * eg2_ref[...], axis=1, keepdims=True) + eb2_ref[...]
    ew = jax.nn.sigmoid(z)
    o_ref[...] = jax.lax.bitcast_convert_type(ew, jnp.int32)


def _score(A, G, Eg2, eb2):
    return pl.pallas_call(
        _k_score,
        grid=(_NBLK,),
        in_specs=[pl.BlockSpec((_RB, 64), lambda i: (i, 0)),
                  pl.BlockSpec((_EB, 128), lambda i: (i, 0)),
                  pl.BlockSpec((1, 64), lambda i: (0, 0)),
                  pl.BlockSpec((1, 1), lambda i: (0, 0))],
        out_specs=pl.BlockSpec((_EB, 1), lambda i: (i, 0)),
        out_shape=jax.ShapeDtypeStruct((_E, 1), jnp.int32),
    )(A, G, Eg2.reshape(1, 64), eb2.reshape(1, 1))


_ROWS = _E // 128


def _k_mask(bits_ref, m_ref, bm_ref):
    bits = bits_ref[...]

    def cnt_ge(v):
        return jnp.sum((bits >= v).astype(jnp.int32))

    def bis_val(_, c):
        lo, hi = c
        mid = lo + (hi - lo + 1) // 2
        ok = cnt_ge(mid) >= _K
        return jnp.where(ok, mid, lo), jnp.where(ok, hi, mid - 1)

    t, _ = jax.lax.fori_loop(0, 31, bis_val,
                             (jnp.int32(0), jnp.int32(0x3F800000)))
    cnt_gt = cnt_ge(t + 1)
    extra = _K - cnt_gt
    tie = bits == t
    idx = (jax.lax.broadcasted_iota(jnp.int32, (_ROWS, 128), 0) * 128
           + jax.lax.broadcasted_iota(jnp.int32, (_ROWS, 128), 1))

    def bis_idx(_, c):
        lo, hi = c
        mid = lo + (hi - lo + 1) // 2
        ok = jnp.sum((tie & (idx <= mid)).astype(jnp.int32)) <= extra
        return jnp.where(ok, mid, lo), jnp.where(ok, hi, mid - 1)

    tau, _ = jax.lax.fori_loop(0, 19, bis_idx,
                               (jnp.int32(-1), jnp.int32(_E - 1)))
    mi = (bits > t) | (tie & (idx <= tau))
    m_ref[...] = mi.astype(jnp.float32)
    sh = jax.lax.broadcasted_iota(jnp.int32, (_ROWS // 32, 32, 128), 1)
    bm_ref[...] = jnp.sum(
        mi.astype(jnp.int32).reshape(_ROWS // 32, 32, 128) << sh, axis=1)


def _mask(bits2d):
    return pl.pallas_call(
        _k_mask,
        in_specs=[pl.BlockSpec((_ROWS, 128), lambda: (0, 0))],
        out_specs=[pl.BlockSpec((_ROWS, 128), lambda: (0, 0)),
                   pl.BlockSpec((_ROWS // 32, 128), lambda: (0, 0))],
        out_shape=[jax.ShapeDtypeStruct((_ROWS, 128), jnp.float32),
                   jax.ShapeDtypeStruct((_ROWS // 32, 128), jnp.int32)],
    )(bits2d)


def _k_final(g_ref, acc_ref, w_ref, dsq_ref, r5_ref, b1_ref, w2_ref, b2_ref,
             fw_ref, fb_ref, pool_ref, fb_out_ref, srow, s0row):
    i = pl.program_id(0)
    bat = i // 4

    @pl.when(i == 0)
    def _init():
        srow[...] = jnp.zeros_like(srow)
        s0row[...] = jnp.zeros_like(s0row)

    flag0 = (i < 4).astype(jnp.float32)
    flagn = (i >= 4).astype(jnp.float32)
    h1 = jnp.maximum(g_ref[...] * flagn + acc_ref[...] * flag0
                     + b1_ref[...], 0.0)
    wsum = jnp.dot(w_ref[...], r5_ref[...],
                   preferred_element_type=jnp.float32)
    coef = wsum + dsq_ref[...] * flag0
    part = jnp.zeros((1, _D), jnp.float32)
    for r in range(8):
        part += jnp.dot(coef[r:r + 1, :], h1[128 * r:128 * (r + 1), :],
                        preferred_element_type=jnp.float32)
    s0row[...] += part

    @pl.when(bat >= 1)
    def _colsum():
        cs = jnp.sum(h1, axis=0, keepdims=True)
        srow[pl.ds(bat, 1), :] += cs

    @pl.when(i == _NBLK - 1)
    def _fin():
        Sm = srow[...]
        Sm = jnp.concatenate([s0row[...], Sm[1:, :]], axis=0)
        pooled = jnp.dot(Sm, w2_ref[...],
                         preferred_element_type=jnp.float32) * (1.0 / _S) \
            + b2_ref[...]
        pool_ref[...] = pooled
        fb_out_ref[...] = jax.nn.sigmoid(
            jnp.dot(pooled, fw_ref[...],
                    preferred_element_type=jnp.float32) + fb_ref[...])


def _final(g, accp, w2d, dsq_nm, b1, W2, b2, Fw, Fb):
    return pl.pallas_call(
        _k_final,
        grid=(_NBLK,),
        in_specs=[pl.BlockSpec((_RB, _D), lambda i: (i, 0)),
                  pl.BlockSpec((_RB, _D), lambda i: (jnp.minimum(i, 3), 0)),
                  pl.BlockSpec((8, 640), lambda i: (i, 0)),
                  pl.BlockSpec((8, 128), lambda i: (jnp.minimum(i, 3), 0)),
                  pl.BlockSpec((640, 128), lambda i: (0, 0)),
                  pl.BlockSpec((1, _D), lambda i: (0, 0)),
                  pl.BlockSpec((_D, _D), lambda i: (0, 0)),
                  pl.BlockSpec((1, _D), lambda i: (0, 0)),
                  pl.BlockSpec((_D, _D), lambda i: (0, 0)),
                  pl.BlockSpec((1, _D), lambda i: (0, 0))],
        out_specs=[pl.BlockSpec((_B, _D), lambda i: (0, 0)),
                   pl.BlockSpec((_B, _D), lambda i: (0, 0))],
        out_shape=[jax.ShapeDtypeStruct((_B, _D), jnp.float32),
                   jax.ShapeDtypeStruct((_B, _D), jnp.float32)],
        scratch_shapes=[pltpu.VMEM((_B, _D), jnp.float32),
                        pltpu.VMEM((1, _D), jnp.float32)],
    )(g, accp, w2d, dsq_nm, jnp.asarray(_FOLD5), b1.reshape(1, _D), W2,
      b2.reshape(1, _D), Fw, Fb.reshape(1, _D))


def kernel(x, W1, b1, W2, b2, Eg1, eb1, Eg2, eb2, Fw, Fb):
    dst = jnp.asarray(_TOPO)
    xf = x.reshape(_N, _D)

    g, A = _dense(xf, W1, Eg1[:_D], eb1)
    Bm = _bmat(xf[:_S], jnp.pad(Eg1[_D:], ((0, 0), (0, 64))))
    G = _sc_gatherB(Bm, dst)
    bits = _score(A, G, Eg2, eb2)
    m2d, bm = _mask(bits.reshape(_ROWS, 128))

    degp = _sc_deg(m2d.reshape(_E), dst)
    dinv_nm, dsq_nm, mdrep_ext = _reduce_deg(degp, m2d)
    dinv0 = dinv_nm.reshape(_S)
    dv = jnp.take(dinv0, dst)
    w = mdrep_ext.reshape(-1)[:_E] * dv
    accp = _sc_acc(bm.reshape(_NBM), jnp.pad(dinv0, (0, 16)), g,
                   jnp.asarray(_EID2), jnp.asarray(_SRC2),
                   jnp.asarray(_DSTG2))
    return _final(g, accp, w.reshape(512, 640), dsq_nm, b1, W2, b2, Fw, Fb)


# speedup vs baseline: 5.4234x; 3.1419x over previous
import functools

import jax
import jax.numpy as jnp
import numpy as np
from jax import lax
from jax.experimental import pallas as pl
from jax.experimental.pallas import tpu as pltpu
from jax.experimental.pallas import tpu_sc as plsc

_B, _S, _D = 16, 4096, 128
_N = _B * _S
_E = _N * 5
_K = int(_E * 0.3)
_RB = 1024
_NBLK = _N // _RB
_EB = _RB * 5


def _tf_core(k1, k2, x1, x2):
    rot = ((13, 15, 26, 6), (17, 29, 16, 24))

    def rotl(x, d):
        return ((x << np.uint32(d)) | (x >> np.uint32(32 - d))).astype(np.uint32)

    ks = [np.uint32(k1), np.uint32(k2),
          np.uint32(np.uint32(k1) ^ np.uint32(k2) ^ np.uint32(0x1BD11BDA))]
    x = [x1.astype(np.uint32) + ks[0], x2.astype(np.uint32) + ks[1]]
    for i in range(5):
        for r in rot[i % 2]:
            x[0] = (x[0] + x[1]).astype(np.uint32)
            x[1] = rotl(x[1], r) ^ x[0]
        x[0] = (x[0] + ks[(i + 1) % 3]).astype(np.uint32)
        x[1] = (x[1] + ks[(i + 2) % 3] + np.uint32(i + 1)).astype(np.uint32)
    return x[0], x[1]


def _np_randint(seed, shape, minval, maxval):
    o1, o2 = _tf_core(0, seed, np.zeros(2, np.uint32),
                      np.arange(2, dtype=np.uint32))
    keys = np.stack([o1, o2], axis=1)
    size = int(np.prod(shape))

    def bits(key):
        b1, b2 = _tf_core(key[0], key[1], np.zeros(size, np.uint32),
                          np.arange(size, dtype=np.uint32))
        return b1 ^ b2

    hi, lo = bits(keys[0]), bits(keys[1])
    span = np.uint32(maxval - minval)
    mult = np.uint32((int(2 ** 16 % span) * int(2 ** 16 % span)) % int(span))
    off = ((hi % span) * mult + (lo % span)) % span
    return (minval + off.astype(np.int64)).astype(np.int32).reshape(shape)


def _topology():
    rnd = _np_randint(42, (_N, 5), 1, _S)
    dst = ((np.arange(_N, dtype=np.int64)[:, None] + rnd) % _S).astype(np.int32)
    return dst.reshape(-1)


_TOPO = _topology()
_FOLD5 = (np.arange(640)[:, None] // 5 == np.arange(128)[None, :]).astype(np.float32)

_NW = 32
_EW = _E // _NW


def _partition():
    tile = _TOPO >> 7
    order = np.argsort(tile, kind="stable").astype(np.int32)
    cnts = np.bincount(tile, minlength=_NW)
    L = int(np.ceil(cnts.max() / 128) * 128)
    eid = np.full((_NW, L), _E, np.int32)
    off = 0
    for t in range(_NW):
        eid[t, :cnts[t]] = order[off:off + cnts[t]]
        off += cnts[t]
    src = np.where(eid < _E, eid // 5, 0).astype(np.int32)
    dstg = np.where(eid < _E, _TOPO[np.minimum(eid, _E - 1)],
                    (np.arange(_NW, dtype=np.int32) * 128 + 128)[:, None])
    return eid, src, dstg.astype(np.int32), L


_EID2, _SRC2, _DSTG2, _LP = _partition()
_NBM = _E // 32


def _sc_acc(bm, dinv0p, g, eid2, src2, dstg2):
    @functools.partial(
        pl.kernel,
        out_type=jax.ShapeDtypeStruct((_S, _D), jnp.float32),
        mesh=_sc_mesh(),
        compiler_params=pltpu.CompilerParams(needs_layout_passes=False),
        scratch_types=[pltpu.VMEM((_NBM + 16,), jnp.int32),
                       pltpu.VMEM((4112,), jnp.float32),
                       pltpu.VMEM((_LP,), jnp.int32),
                       pltpu.VMEM((_LP,), jnp.int32),
                       pltpu.VMEM((_LP,), jnp.int32),
                       pltpu.VMEM((_LP + 16,), jnp.int32),
                       pltpu.VMEM((_LP + 16,), jnp.int32),
                       pltpu.VMEM((_LP + 16,), jnp.float32),
                       pltpu.VMEM((128, _D), jnp.float32),
                       pltpu.VMEM((129, _D), jnp.float32),
                       pltpu.SemaphoreType.DMA],
    )
    def k(bm_hbm, dinv_hbm, g_hbm, eid_hbm, src_hbm, dstg_hbm, acc_hbm,
          bm_v, dinv_v, eid_v, src_v, dstg_v, csrc_v, crow_v, cw_v,
          grow_v, acc_v, sem):
        wid = lax.axis_index("s") * 2 + lax.axis_index("c")
        base = wid * 128
        iota = lax.iota(jnp.int32, 16)

        pltpu.sync_copy(bm_hbm.at[pl.ds(0, _NBM)], bm_v.at[pl.ds(0, _NBM)])
        pltpu.sync_copy(dinv_hbm, dinv_v)
        pltpu.sync_copy(eid_hbm.at[wid], eid_v)
        pltpu.sync_copy(src_hbm.at[wid], src_v)
        pltpu.sync_copy(dstg_hbm.at[wid], dstg_v)
        pltpu.async_copy(g_hbm.at[pl.ds(base, 128)], grow_v, sem).wait()

        def self_loop(r, c):
            d = plsc.load_gather(dinv_v, [jnp.full((16,), base, jnp.int32) + r])
            d2 = d * d
            for v in range(8):
                gval = plsc.load_gather(
                    grow_v, [jnp.full((16,), r, jnp.int32), v * 16 + iota])
                plsc.store_scatter(
                    acc_v, [jnp.full((16,), r, jnp.int32), v * 16 + iota],
                    gval * d2)
            return c

        lax.fori_loop(0, 128, self_loop, 0)

        def fill(i, c):
            csrc_v[pl.ds(i * 16, 16)] = jnp.zeros((16,), jnp.int32)
            crow_v[pl.ds(i * 16, 16)] = jnp.full((16,), 128, jnp.int32)
            cw_v[pl.ds(i * 16, 16)] = jnp.zeros((16,), jnp.float32)
            return c

        lax.fori_loop(0, (_LP + 16) // 16, fill, 0)

        def groups(gi, kcnt):
            eid = eid_v[pl.ds(gi * 16, 16)]
            wi = ((eid >> 12) << 7) | (eid & 127)
            bit = (eid >> 7) & 31
            word = plsc.load_gather(bm_v, [wi])
            mb = ((word >> bit) & 1).astype(jnp.float32)
            srci = src_v[pl.ds(gi * 16, 16)]
            dstg = dstg_v[pl.ds(gi * 16, 16)]
            dfull = jnp.where(
                srci < _S,
                plsc.load_gather(dinv_v, [jnp.minimum(srci, _S - 1)]), 1.0)
            w = mb * dfull * plsc.load_gather(dinv_v, [dstg])
            keep = w != 0.0
            plsc.store_compressed(csrc_v.at[pl.ds(kcnt, 16)], srci, mask=keep)
            plsc.store_compressed(crow_v.at[pl.ds(kcnt, 16)], dstg - base, mask=keep)
            plsc.store_compressed(cw_v.at[pl.ds(kcnt, 16)], w, mask=keep)
            return kcnt + jnp.sum(keep.astype(jnp.int32))

        kcnt = lax.fori_loop(0, _LP // 16, groups, jnp.int32(0))

        def chunk(c2, c):
            pltpu.async_copy(
                g_hbm.at[csrc_v.at[pl.ds(c2 * 128, 128)]], grow_v, sem).wait()

            def edge(e2, c3):
                sp = jnp.full((16,), c2 * 128, jnp.int32) + e2
                wspl = plsc.load_gather(cw_v, [sp])
                rowspl = plsc.load_gather(crow_v, [sp])
                espl = jnp.full((16,), e2, jnp.int32)
                for v in range(8):
                    gval = plsc.load_gather(grow_v, [espl, v * 16 + iota])
                    plsc.addupdate_scatter(
                        acc_v, [rowspl, v * 16 + iota], gval * wspl)
                return c3

            lax.fori_loop(0, 128, edge, 0)
            return c

        lax.fori_loop(0, (kcnt + 127) // 128, chunk, 0)
        pltpu.sync_copy(acc_v.at[pl.ds(0, 128), :], acc_hbm.at[pl.ds(base, 128)])

    return k(bm, dinv0p, g, eid2, src2, dstg2)


def _sc_mesh():
    return plsc.VectorSubcoreMesh(core_axis_name="c", subcore_axis_name="s",
                                  num_cores=2, num_subcores=16)


def _sc_gatherB(Bm, dstarr):
    @functools.partial(
        pl.kernel,
        out_type=jax.ShapeDtypeStruct((_E, 128), jnp.float32),
        mesh=_sc_mesh(),
        scratch_types=[pltpu.VMEM((512,), jnp.int32),
                       pltpu.VMEM((512, 128), jnp.float32),
                       pltpu.SemaphoreType.DMA],
    )
    def k(b_hbm, dst_hbm, g_hbm, idx_v, rows_v, sem):
        wid = lax.axis_index("s") * 2 + lax.axis_index("c")
        base = wid * _EW

        def chunk(ci, carry):
            row0 = base + ci * 512
            pltpu.sync_copy(dst_hbm.at[pl.ds(row0, 512)], idx_v)
            cps = [pltpu.async_copy(b_hbm.at[idx_v.at[pl.ds(kk * 128, 128)]],
                                    rows_v.at[pl.ds(kk * 128, 128), :], sem)
                   for kk in range(4)]
            for cp in cps:
                cp.wait()
            pltpu.sync_copy(rows_v, g_hbm.at[pl.ds(row0, 512)])
            return carry

        lax.fori_loop(0, _EW // 512, chunk, 0)

    return k(Bm, dstarr)


def _sc_deg(m, dstarr):
    @functools.partial(
        pl.kernel,
        out_type=jax.ShapeDtypeStruct((_NW, _S), jnp.float32),
        mesh=_sc_mesh(),
        compiler_params=pltpu.CompilerParams(needs_layout_passes=False),
        scratch_types=[pltpu.VMEM((2048,), jnp.float32),
                       pltpu.VMEM((2048,), jnp.int32),
                       pltpu.VMEM((_S,), jnp.float32)],
    )
    def k(m_hbm, dst_hbm, out_hbm, m_v, dst_v, deg_v):
        wid = lax.axis_index("s") * 2 + lax.axis_index("c")
        base = wid * _EW
        zero16 = jnp.zeros((16,), jnp.float32)

        def z(i, c):
            deg_v[pl.ds(i * 16, 16)] = zero16
            return c

        lax.fori_loop(0, _S // 16, z, 0)

        def chunk(ci, c):
            pltpu.sync_copy(m_hbm.at[pl.ds(base + ci * 2048, 2048)], m_v)
            pltpu.sync_copy(dst_hbm.at[pl.ds(base + ci * 2048, 2048)], dst_v)

            def inner(kk, c2):
                idx = dst_v[pl.ds(kk * 16, 16)]
                vals = m_v[pl.ds(kk * 16, 16)]
                plsc.addupdate_scatter(deg_v, [idx], vals)
                return c2

            lax.fori_loop(0, 128, inner, 0)
            return c

        lax.fori_loop(0, _EW // 2048, chunk, 0)
        pltpu.sync_copy(deg_v, out_hbm.at[wid])

    return k(m, dstarr)


def _sc_w(mdrep, dstarr, dinv0p):
    @functools.partial(
        pl.kernel,
        out_type=jax.ShapeDtypeStruct((_E,), jnp.float32),
        mesh=_sc_mesh(),
        compiler_params=pltpu.CompilerParams(needs_layout_passes=False),
        scratch_types=[pltpu.VMEM((2048,), jnp.float32),
                       pltpu.VMEM((2048,), jnp.int32),
                       pltpu.VMEM((2048,), jnp.float32),
                       pltpu.VMEM((4112,), jnp.float32)],
    )
    def k(md_hbm, dst_hbm, dinv_hbm, w_hbm, md_v, dst_v, w_v, dinv_v):
        wid = lax.axis_index("s") * 2 + lax.axis_index("c")
        base = wid * _EW
        pltpu.sync_copy(dinv_hbm, dinv_v)

        def chunk(ci, c):
            off = base + ci * 2048
            pltpu.sync_copy(md_hbm.at[pl.ds(off, 2048)], md_v)
            pltpu.sync_copy(dst_hbm.at[pl.ds(off, 2048)], dst_v)

            def inner(kk, c2):
                dstg = dst_v[pl.ds(kk * 16, 16)]
                md = md_v[pl.ds(kk * 16, 16)]
                w_v[pl.ds(kk * 16, 16)] = md * plsc.load_gather(dinv_v, [dstg])
                return c2

            lax.fori_loop(0, 128, inner, 0)
            pltpu.sync_copy(w_v, w_hbm.at[pl.ds(off, 2048)])
            return c

        lax.fori_loop(0, _EW // 2048, chunk, 0)

    return k(mdrep, dstarr, dinv0p)


def _k_reduce(degp_ref, m_ref, rep_ref, dinv_ref, dsq_ref, mdrep_ref):
    deg = 1.0 + jnp.sum(degp_ref[...], axis=0)
    dinv = jax.lax.rsqrt(deg)
    dinv_ref[...] = dinv
    dsq_ref[...] = dinv * dinv
    dfull = jnp.concatenate(
        [dinv, jnp.ones((480, 128), jnp.float32)], axis=0)
    dinvrep = jnp.dot(dfull, rep_ref[...],
                      preferred_element_type=jnp.float32)
    md = m_ref[...] * dinvrep
    mdrep_ref[...] = jnp.concatenate(
        [md, jnp.zeros((1, 640), jnp.float32)], axis=0)


def _reduce_deg(degp, m2d):
    return pl.pallas_call(
        _k_reduce,
        in_specs=[pl.BlockSpec((_NW, 32, 128), lambda: (0, 0, 0)),
                  pl.BlockSpec((512, 640), lambda: (0, 0)),
                  pl.BlockSpec((128, 640), lambda: (0, 0))],
        out_specs=[pl.BlockSpec((32, 128), lambda: (0, 0)),
                   pl.BlockSpec((32, 128), lambda: (0, 0)),
                   pl.BlockSpec((513, 640), lambda: (0, 0))],
        out_shape=[jax.ShapeDtypeStruct((32, 128), jnp.float32),
                   jax.ShapeDtypeStruct((32, 128), jnp.float32),
                   jax.ShapeDtypeStruct((513, 640), jnp.float32)],
    )(degp.reshape(_NW, 32, 128), m2d.reshape(512, 640),
      jnp.asarray(_FOLD5.T))


def _k_dense(x_ref, w1_ref, e1t_ref, eb1_ref, g_ref, a_ref):
    xb = x_ref[...]
    g_ref[...] = jnp.dot(xb, w1_ref[...], preferred_element_type=jnp.float32)
    a_ref[...] = jnp.dot(xb, e1t_ref[...],
                         preferred_element_type=jnp.float32) + eb1_ref[...]


def _dense(xf, W1, Eg1t, eb1):
    return pl.pallas_call(
        _k_dense,
        grid=(_NBLK,),
        in_specs=[pl.BlockSpec((_RB, _D), lambda i: (i, 0)),
                  pl.BlockSpec((_D, _D), lambda i: (0, 0)),
                  pl.BlockSpec((_D, 64), lambda i: (0, 0)),
                  pl.BlockSpec((1, 64), lambda i: (0, 0))],
        out_specs=[pl.BlockSpec((_RB, _D), lambda i: (i, 0)),
                   pl.BlockSpec((_RB, 64), lambda i: (i, 0))],
        out_shape=[jax.ShapeDtypeStruct((_N, _D), jnp.float32),
                   jax.ShapeDtypeStruct((_N, 64), jnp.float32)],
    )(xf, W1, Eg1t, eb1.reshape(1, 64))


def _k_small_mm(x_ref, w_ref, o_ref):
    o_ref[...] = jnp.dot(x_ref[...], w_ref[...],
                         preferred_element_type=jnp.float32)


def _bmat(x0, Eg1b):
    return pl.pallas_call(
        _k_small_mm,
        grid=(4,),
        in_specs=[pl.BlockSpec((_RB, _D), lambda i: (i, 0)),
                  pl.BlockSpec((_D, _D), lambda i: (0, 0))],
        out_specs=pl.BlockSpec((_RB, _D), lambda i: (i, 0)),
        out_shape=jax.ShapeDtypeStruct((_S, _D), jnp.float32),
    )(x0, Eg1b)


def _k_score(a_ref, gth_ref, eg2_ref, eb2_ref, o_ref):
    a = a_ref[...]
    ar = jnp.broadcast_to(a[:, None, :], (_RB, 5, 64)).reshape(_EB, 64)
    r = jnp.maximum(ar + gth_ref[...][:, :64], 0.0)
    z = jnp.sum(r * eg2_ref[...], axis=1, keepdims=True) + eb2_ref[...]
    ew = jax.nn.sigmoid(z)
    o_ref[...] = jax.lax.bitcast_convert_type(ew, jnp.int32)


def _score(A, G, Eg2, eb2):
    return pl.pallas_call(
        _k_score,
        grid=(_NBLK,),
        in_specs=[pl.BlockSpec((_RB, 64), lambda i: (i, 0)),
                  pl.BlockSpec((_EB, 128), lambda i: (i, 0)),
                  pl.BlockSpec((1, 64), lambda i: (0, 0)),
                  pl.BlockSpec((1, 1), lambda i: (0, 0))],
        out_specs=pl.BlockSpec((_EB, 1), lambda i: (i, 0)),
        out_shape=jax.ShapeDtypeStruct((_E, 1), jnp.int32),
    )(A, G, Eg2.reshape(1, 64), eb2.reshape(1, 1))


_ROWS = _E // 128


def _k_mask(bits_ref, m_ref, bm_ref):
    bits = bits_ref[...]

    def cnt_ge(v):
        return jnp.sum((bits >= v).astype(jnp.int32))

    def bis_val(_, c):
        lo, hi = c
        mid = lo + (hi - lo + 1) // 2
        ok = cnt_ge(mid) >= _K
        return jnp.where(ok, mid, lo), jnp.where(ok, hi, mid - 1)

    t, _ = jax.lax.fori_loop(0, 31, bis_val,
                             (jnp.int32(0), jnp.int32(0x3F800000)))
    cnt_gt = cnt_ge(t + 1)
    extra = _K - cnt_gt
    tie = bits == t
    idx = (jax.lax.broadcasted_iota(jnp.int32, (_ROWS, 128), 0) * 128
           + jax.lax.broadcasted_iota(jnp.int32, (_ROWS, 128), 1))

    def bis_idx(_, c):
        lo, hi = c
        mid = lo + (hi - lo + 1) // 2
        ok = jnp.sum((tie & (idx <= mid)).astype(jnp.int32)) <= extra
        return jnp.where(ok, mid, lo), jnp.where(ok, hi, mid - 1)

    tau, _ = jax.lax.fori_loop(0, 19, bis_idx,
                               (jnp.int32(-1), jnp.int32(_E - 1)))
    mi = (bits > t) | (tie & (idx <= tau))
    m_ref[...] = mi.astype(jnp.float32)
    sh = jax.lax.broadcasted_iota(jnp.int32, (_ROWS // 32, 32, 128), 1)
    bm_ref[...] = jnp.sum(
        mi.astype(jnp.int32).reshape(_ROWS // 32, 32, 128) << sh, axis=1)


def _mask(bits2d):
    return pl.pallas_call(
        _k_mask,
        in_specs=[pl.BlockSpec((_ROWS, 128), lambda: (0, 0))],
        out_specs=[pl.BlockSpec((_ROWS, 128), lambda: (0, 0)),
                   pl.BlockSpec((_ROWS // 32, 128), lambda: (0, 0))],
        out_shape=[jax.ShapeDtypeStruct((_ROWS, 128), jnp.float32),
                   jax.ShapeDtypeStruct((_ROWS // 32, 128), jnp.int32)],
    )(bits2d)


def _k_final(g_ref, acc_ref, w_ref, dsq_ref, r5_ref, b1_ref, w2_ref, b2_ref,
             fw_ref, fb_ref, pool_ref, fb_out_ref, srow, s0row):
    i = pl.program_id(0)
    bat = i // 4

    @pl.when(i == 0)
    def _init():
        srow[...] = jnp.zeros_like(srow)
        s0row[...] = jnp.zeros_like(s0row)

    flag0 = (i < 4).astype(jnp.float32)
    flagn = (i >= 4).astype(jnp.float32)
    h1 = jnp.maximum(g_ref[...] * flagn + acc_ref[...] * flag0
                     + b1_ref[...], 0.0)
    wsum = jnp.dot(w_ref[...], r5_ref[...],
                   preferred_element_type=jnp.float32)
    coef = wsum + dsq_ref[...] * flag0
    part = jnp.zeros((1, _D), jnp.float32)
    for r in range(8):
        part += jnp.dot(coef[r:r + 1, :], h1[128 * r:128 * (r + 1), :],
                        preferred_element_type=jnp.float32)
    s0row[...] += part

    @pl.when(bat >= 1)
    def _colsum():
        cs = jnp.sum(h1, axis=0, keepdims=True)
        srow[pl.ds(bat, 1), :] += cs

    @pl.when(i == _NBLK - 1)
    def _fin():
        Sm = srow[...]
        Sm = jnp.concatenate([s0row[...], Sm[1:, :]], axis=0)
        pooled = jnp.dot(Sm, w2_ref[...],
                         preferred_element_type=jnp.float32) * (1.0 / _S) \
            + b2_ref[...]
        pool_ref[...] = pooled
        fb_out_ref[...] = jax.nn.sigmoid(
            jnp.dot(pooled, fw_ref[...],
                    preferred_element_type=jnp.float32) + fb_ref[...])


def _final(g, accp, w2d, dsq_nm, b1, W2, b2, Fw, Fb):
    return pl.pallas_call(
        _k_final,
        grid=(_NBLK,),
        in_specs=[pl.BlockSpec((_RB, _D), lambda i: (i, 0)),
                  pl.BlockSpec((_RB, _D), lambda i: (jnp.minimum(i, 3), 0)),
                  pl.BlockSpec((8, 640), lambda i: (i, 0)),
                  pl.BlockSpec((8, 128), lambda i: (jnp.minimum(i, 3), 0)),
                  pl.BlockSpec((640, 128), lambda i: (0, 0)),
                  pl.BlockSpec((1, _D), lambda i: (0, 0)),
                  pl.BlockSpec((_D, _D), lambda i: (0, 0)),
                  pl.BlockSpec((1, _D), lambda i: (0, 0)),
                  pl.BlockSpec((_D, _D), lambda i: (0, 0)),
                  pl.BlockSpec((1, _D), lambda i: (0, 0))],
        out_specs=[pl.BlockSpec((_B, _D), lambda i: (0, 0)),
                   pl.BlockSpec((_B, _D), lambda i: (0, 0))],
        out_shape=[jax.ShapeDtypeStruct((_B, _D), jnp.float32),
                   jax.ShapeDtypeStruct((_B, _D), jnp.float32)],
        scratch_shapes=[pltpu.VMEM((_B, _D), jnp.float32),
                        pltpu.VMEM((1, _D), jnp.float32)],
    )(g, accp, w2d, dsq_nm, jnp.asarray(_FOLD5), b1.reshape(1, _D), W2,
      b2.reshape(1, _D), Fw, Fb.reshape(1, _D))


def kernel(x, W1, b1, W2, b2, Eg1, eb1, Eg2, eb2, Fw, Fb):
    dst = jnp.asarray(_TOPO)
    xf = x.reshape(_N, _D)

    g, A = _dense(xf, W1, Eg1[:_D], eb1)
    Bm = _bmat(xf[:_S], jnp.pad(Eg1[_D:], ((0, 0), (0, 64))))
    G = _sc_gatherB(Bm, dst)
    bits = _score(A, G, Eg2, eb2)
    m2d, bm = _mask(bits.reshape(_ROWS, 128))

    degp = _sc_deg(m2d.reshape(_E), dst)
    dinv_nm, dsq_nm, mdrep_ext = _reduce_deg(degp, m2d)
    dinv0 = dinv_nm.reshape(_S)
    dinv0p = jnp.pad(dinv0, (0, 16))
    w = _sc_w(mdrep_ext.reshape(-1)[:_E], dst, dinv0p)
    accp = _sc_acc(bm.reshape(_NBM), dinv0p, g,
                   jnp.asarray(_EID2), jnp.asarray(_SRC2),
                   jnp.asarray(_DSTG2))
    return _final(g, accp, w.reshape(512, 640), dsq_nm, b1, W2, b2, Fw, Fb)


# speedup vs baseline: 5.6524x; 1.0422x over previous
import functools

import jax
import jax.numpy as jnp
import numpy as np
from jax import lax
from jax.experimental import pallas as pl
from jax.experimental.pallas import tpu as pltpu
from jax.experimental.pallas import tpu_sc as plsc

_B, _S, _D = 16, 4096, 128
_N = _B * _S
_E = _N * 5
_K = int(_E * 0.3)
_RB = 1024
_NBLK = _N // _RB
_EB = _RB * 5


def _tf_core(k1, k2, x1, x2):
    rot = ((13, 15, 26, 6), (17, 29, 16, 24))

    def rotl(x, d):
        return ((x << np.uint32(d)) | (x >> np.uint32(32 - d))).astype(np.uint32)

    ks = [np.uint32(k1), np.uint32(k2),
          np.uint32(np.uint32(k1) ^ np.uint32(k2) ^ np.uint32(0x1BD11BDA))]
    x = [x1.astype(np.uint32) + ks[0], x2.astype(np.uint32) + ks[1]]
    for i in range(5):
        for r in rot[i % 2]:
            x[0] = (x[0] + x[1]).astype(np.uint32)
            x[1] = rotl(x[1], r) ^ x[0]
        x[0] = (x[0] + ks[(i + 1) % 3]).astype(np.uint32)
        x[1] = (x[1] + ks[(i + 2) % 3] + np.uint32(i + 1)).astype(np.uint32)
    return x[0], x[1]


def _np_randint(seed, shape, minval, maxval):
    o1, o2 = _tf_core(0, seed, np.zeros(2, np.uint32),
                      np.arange(2, dtype=np.uint32))
    keys = np.stack([o1, o2], axis=1)
    size = int(np.prod(shape))

    def bits(key):
        b1, b2 = _tf_core(key[0], key[1], np.zeros(size, np.uint32),
                          np.arange(size, dtype=np.uint32))
        return b1 ^ b2

    hi, lo = bits(keys[0]), bits(keys[1])
    span = np.uint32(maxval - minval)
    mult = np.uint32((int(2 ** 16 % span) * int(2 ** 16 % span)) % int(span))
    off = ((hi % span) * mult + (lo % span)) % span
    return (minval + off.astype(np.int64)).astype(np.int32).reshape(shape)


def _topology():
    rnd = _np_randint(42, (_N, 5), 1, _S)
    dst = ((np.arange(_N, dtype=np.int64)[:, None] + rnd) % _S).astype(np.int32)
    return dst.reshape(-1)


_TOPO = _topology()
_FOLD5 = (np.arange(640)[:, None] // 5 == np.arange(128)[None, :]).astype(np.float32)

_NW = 32
_EW = _E // _NW


def _partition():
    tile = _TOPO >> 7
    order = np.argsort(tile, kind="stable").astype(np.int32)
    cnts = np.bincount(tile, minlength=_NW)
    L = int(np.ceil(cnts.max() / 128) * 128)
    eid = np.full((_NW, L), _E, np.int32)
    off = 0
    for t in range(_NW):
        eid[t, :cnts[t]] = order[off:off + cnts[t]]
        off += cnts[t]
    src = np.where(eid < _E, eid // 5, 0).astype(np.int32)
    dstg = np.where(eid < _E, _TOPO[np.minimum(eid, _E - 1)],
                    (np.arange(_NW, dtype=np.int32) * 128 + 128)[:, None])
    return eid, src, dstg.astype(np.int32), L


_EID2, _SRC2, _DSTG2, _LP = _partition()
_NBM = _E // 32


def _sc_acc(bm, dinv0p, g, eid2, src2, dstg2):
    @functools.partial(
        pl.kernel,
        out_type=jax.ShapeDtypeStruct((_S, _D), jnp.float32),
        mesh=_sc_mesh(),
        compiler_params=pltpu.CompilerParams(needs_layout_passes=False),
        scratch_types=[pltpu.VMEM((_NBM + 16,), jnp.int32),
                       pltpu.VMEM((4112,), jnp.float32),
                       pltpu.VMEM((_LP,), jnp.int32),
                       pltpu.VMEM((_LP,), jnp.int32),
                       pltpu.VMEM((_LP,), jnp.int32),
                       pltpu.VMEM((_LP + 16,), jnp.int32),
                       pltpu.VMEM((_LP + 16,), jnp.int32),
                       pltpu.VMEM((_LP + 16,), jnp.float32),
                       pltpu.VMEM((128, _D), jnp.float32),
                       pltpu.VMEM((129, _D), jnp.float32),
                       pltpu.SemaphoreType.DMA],
    )
    def k(bm_hbm, dinv_hbm, g_hbm, eid_hbm, src_hbm, dstg_hbm, acc_hbm,
          bm_v, dinv_v, eid_v, src_v, dstg_v, csrc_v, crow_v, cw_v,
          grow_v, acc_v, sem):
        wid = lax.axis_index("s") * 2 + lax.axis_index("c")
        base = wid * 128
        iota = lax.iota(jnp.int32, 16)

        pltpu.sync_copy(bm_hbm.at[pl.ds(0, _NBM)], bm_v.at[pl.ds(0, _NBM)])
        pltpu.sync_copy(dinv_hbm, dinv_v)
        pltpu.sync_copy(eid_hbm.at[wid], eid_v)
        pltpu.sync_copy(src_hbm.at[wid], src_v)
        pltpu.sync_copy(dstg_hbm.at[wid], dstg_v)
        pltpu.async_copy(g_hbm.at[pl.ds(base, 128)], grow_v, sem).wait()

        def self_loop(r, c):
            d = plsc.load_gather(dinv_v, [jnp.full((16,), base, jnp.int32) + r])
            d2 = d * d
            for v in range(8):
                gval = plsc.load_gather(
                    grow_v, [jnp.full((16,), r, jnp.int32), v * 16 + iota])
                plsc.store_scatter(
                    acc_v, [jnp.full((16,), r, jnp.int32), v * 16 + iota],
                    gval * d2)
            return c

        lax.fori_loop(0, 128, self_loop, 0)

        def fill(i, c):
            csrc_v[pl.ds(i * 16, 16)] = jnp.zeros((16,), jnp.int32)
            crow_v[pl.ds(i * 16, 16)] = jnp.full((16,), 128, jnp.int32)
            cw_v[pl.ds(i * 16, 16)] = jnp.zeros((16,), jnp.float32)
            return c

        lax.fori_loop(0, (_LP + 16) // 16, fill, 0)

        def groups(gi, kcnt):
            eid = eid_v[pl.ds(gi * 16, 16)]
            wi = ((eid >> 12) << 7) | (eid & 127)
            bit = (eid >> 7) & 31
            word = plsc.load_gather(bm_v, [wi])
            mb = ((word >> bit) & 1).astype(jnp.float32)
            srci = src_v[pl.ds(gi * 16, 16)]
            dstg = dstg_v[pl.ds(gi * 16, 16)]
            dfull = jnp.where(
                srci < _S,
                plsc.load_gather(dinv_v, [jnp.minimum(srci, _S - 1)]), 1.0)
            w = mb * dfull * plsc.load_gather(dinv_v, [dstg])
            keep = w != 0.0
            plsc.store_compressed(csrc_v.at[pl.ds(kcnt, 16)], srci, mask=keep)
            plsc.store_compressed(crow_v.at[pl.ds(kcnt, 16)], dstg - base, mask=keep)
            plsc.store_compressed(cw_v.at[pl.ds(kcnt, 16)], w, mask=keep)
            return kcnt + jnp.sum(keep.astype(jnp.int32))

        kcnt = lax.fori_loop(0, _LP // 16, groups, jnp.int32(0))

        def chunk(c2, c):
            pltpu.async_copy(
                g_hbm.at[csrc_v.at[pl.ds(c2 * 128, 128)]], grow_v, sem).wait()

            def edge(e2, c3):
                sp = jnp.full((16,), c2 * 128, jnp.int32) + e2
                wspl = plsc.load_gather(cw_v, [sp])
                rowspl = plsc.load_gather(crow_v, [sp])
                espl = jnp.full((16,), e2, jnp.int32)
                for v in range(8):
                    gval = plsc.load_gather(grow_v, [espl, v * 16 + iota])
                    plsc.addupdate_scatter(
                        acc_v, [rowspl, v * 16 + iota], gval * wspl)
                return c3

            lax.fori_loop(0, 128, edge, 0)
            return c

        lax.fori_loop(0, (kcnt + 127) // 128, chunk, 0)
        pltpu.sync_copy(acc_v.at[pl.ds(0, 128), :], acc_hbm.at[pl.ds(base, 128)])

    return k(bm, dinv0p, g, eid2, src2, dstg2)


def _sc_mesh():
    return plsc.VectorSubcoreMesh(core_axis_name="c", subcore_axis_name="s",
                                  num_cores=2, num_subcores=16)


def _sc_gatherB(Bm, dstarr):
    @functools.partial(
        pl.kernel,
        out_type=jax.ShapeDtypeStruct((_E, 128), jnp.float32),
        mesh=_sc_mesh(),
        scratch_types=[pltpu.VMEM((512,), jnp.int32),
                       pltpu.VMEM((512, 128), jnp.float32),
                       pltpu.SemaphoreType.DMA],
    )
    def k(b_hbm, dst_hbm, g_hbm, idx_v, rows_v, sem):
        wid = lax.axis_index("s") * 2 + lax.axis_index("c")
        base = wid * _EW

        def chunk(ci, carry):
            row0 = base + ci * 512
            pltpu.sync_copy(dst_hbm.at[pl.ds(row0, 512)], idx_v)
            cps = [pltpu.async_copy(b_hbm.at[idx_v.at[pl.ds(kk * 128, 128)]],
                                    rows_v.at[pl.ds(kk * 128, 128), :], sem)
                   for kk in range(4)]
            for cp in cps:
                cp.wait()
            pltpu.sync_copy(rows_v, g_hbm.at[pl.ds(row0, 512)])
            return carry

        lax.fori_loop(0, _EW // 512, chunk, 0)

    return k(Bm, dstarr)


def _sc_deg(m, dstarr):
    @functools.partial(
        pl.kernel,
        out_type=jax.ShapeDtypeStruct((_NW, _S), jnp.float32),
        mesh=_sc_mesh(),
        compiler_params=pltpu.CompilerParams(needs_layout_passes=False),
        scratch_types=[pltpu.VMEM((2048,), jnp.float32),
                       pltpu.VMEM((2048,), jnp.int32),
                       pltpu.VMEM((_S,), jnp.float32)],
    )
    def k(m_hbm, dst_hbm, out_hbm, m_v, dst_v, deg_v):
        wid = lax.axis_index("s") * 2 + lax.axis_index("c")
        base = wid * _EW
        zero16 = jnp.zeros((16,), jnp.float32)

        def z(i, c):
            deg_v[pl.ds(i * 16, 16)] = zero16
            return c

        lax.fori_loop(0, _S // 16, z, 0)

        def chunk(ci, c):
            pltpu.sync_copy(m_hbm.at[pl.ds(base + ci * 2048, 2048)], m_v)
            pltpu.sync_copy(dst_hbm.at[pl.ds(base + ci * 2048, 2048)], dst_v)

            def inner(kk, c2):
                idx = dst_v[pl.ds(kk * 16, 16)]
                vals = m_v[pl.ds(kk * 16, 16)]
                plsc.addupdate_scatter(deg_v, [idx], vals)
                return c2

            lax.fori_loop(0, 128, inner, 0)
            return c

        lax.fori_loop(0, _EW // 2048, chunk, 0)
        pltpu.sync_copy(deg_v, out_hbm.at[wid])

    return k(m, dstarr)


def _sc_w(mdrep, dstarr, dinv0p):
    @functools.partial(
        pl.kernel,
        out_type=jax.ShapeDtypeStruct((_E,), jnp.float32),
        mesh=_sc_mesh(),
        compiler_params=pltpu.CompilerParams(needs_layout_passes=False),
        scratch_types=[pltpu.VMEM((2048,), jnp.float32),
                       pltpu.VMEM((2048,), jnp.int32),
                       pltpu.VMEM((2048,), jnp.float32),
                       pltpu.VMEM((4112,), jnp.float32)],
    )
    def k(md_hbm, dst_hbm, dinv_hbm, w_hbm, md_v, dst_v, w_v, dinv_v):
        wid = lax.axis_index("s") * 2 + lax.axis_index("c")
        base = wid * _EW
        pltpu.sync_copy(dinv_hbm, dinv_v)

        def chunk(ci, c):
            off = base + ci * 2048
            pltpu.sync_copy(md_hbm.at[pl.ds(off, 2048)], md_v)
            pltpu.sync_copy(dst_hbm.at[pl.ds(off, 2048)], dst_v)

            def inner(kk, c2):
                dstg = dst_v[pl.ds(kk * 16, 16)]
                md = md_v[pl.ds(kk * 16, 16)]
                w_v[pl.ds(kk * 16, 16)] = md * plsc.load_gather(dinv_v, [dstg])
                return c2

            lax.fori_loop(0, 128, inner, 0)
            pltpu.sync_copy(w_v, w_hbm.at[pl.ds(off, 2048)])
            return c

        lax.fori_loop(0, _EW // 2048, chunk, 0)

    return k(mdrep, dstarr, dinv0p)


def _k_reduce(degp_ref, m_ref, rep_ref, dinv_ref, dsq_ref, mdrep_ref):
    deg = 1.0 + jnp.sum(degp_ref[...], axis=0)
    dinv = jax.lax.rsqrt(deg)
    dinv_ref[...] = dinv
    dsq_ref[...] = dinv * dinv
    dfull = jnp.concatenate(
        [dinv, jnp.ones((480, 128), jnp.float32)], axis=0)
    dinvrep = jnp.dot(dfull, rep_ref[...],
                      preferred_element_type=jnp.float32)
    md = m_ref[...] * dinvrep
    mdrep_ref[...] = jnp.concatenate(
        [md, jnp.zeros((1, 640), jnp.float32)], axis=0)


def _reduce_deg(degp, m2d):
    return pl.pallas_call(
        _k_reduce,
        in_specs=[pl.BlockSpec((_NW, 32, 128), lambda: (0, 0, 0)),
                  pl.BlockSpec((512, 640), lambda: (0, 0)),
                  pl.BlockSpec((128, 640), lambda: (0, 0))],
        out_specs=[pl.BlockSpec((32, 128), lambda: (0, 0)),
                   pl.BlockSpec((32, 128), lambda: (0, 0)),
                   pl.BlockSpec((513, 640), lambda: (0, 0))],
        out_shape=[jax.ShapeDtypeStruct((32, 128), jnp.float32),
                   jax.ShapeDtypeStruct((32, 128), jnp.float32),
                   jax.ShapeDtypeStruct((513, 640), jnp.float32)],
    )(degp.reshape(_NW, 32, 128), m2d.reshape(512, 640),
      jnp.asarray(_FOLD5.T))


def _k_dense(x_ref, w1_ref, e1t_ref, eb1_ref, g_ref, a_ref):
    xb = x_ref[...]
    g_ref[...] = jnp.dot(xb, w1_ref[...], preferred_element_type=jnp.float32)
    a_ref[...] = jnp.dot(xb, e1t_ref[...],
                         preferred_element_type=jnp.float32) + eb1_ref[...]


def _dense(xf, W1, Eg1t, eb1):
    return pl.pallas_call(
        _k_dense,
        grid=(_NBLK,),
        in_specs=[pl.BlockSpec((_RB, _D), lambda i: (i, 0)),
                  pl.BlockSpec((_D, _D), lambda i: (0, 0)),
                  pl.BlockSpec((_D, 64), lambda i: (0, 0)),
                  pl.BlockSpec((1, 64), lambda i: (0, 0))],
        out_specs=[pl.BlockSpec((_RB, _D), lambda i: (i, 0)),
                   pl.BlockSpec((_RB, 64), lambda i: (i, 0))],
        out_shape=[jax.ShapeDtypeStruct((_N, _D), jnp.float32),
                   jax.ShapeDtypeStruct((_N, 64), jnp.float32)],
    )(xf, W1, Eg1t, eb1.reshape(1, 64))


def _k_small_mm(x_ref, w_ref, o_ref):
    o_ref[...] = jnp.dot(x_ref[...], w_ref[...],
                         preferred_element_type=jnp.float32)


def _bmat(x0, Eg1b):
    return pl.pallas_call(
        _k_small_mm,
        grid=(4,),
        in_specs=[pl.BlockSpec((_RB, _D), lambda i: (i, 0)),
                  pl.BlockSpec((_D, _D), lambda i: (0, 0))],
        out_specs=pl.BlockSpec((_RB, _D), lambda i: (i, 0)),
        out_shape=jax.ShapeDtypeStruct((_S, _D), jnp.float32),
    )(x0, Eg1b)


def _k_score(a_ref, gth_ref, eg2_ref, eb2_ref, o_ref):
    a = a_ref[...]
    ar = jnp.broadcast_to(a[:, None, :], (_RB, 5, 64)).reshape(_EB, 64)
    r = jnp.maximum(ar + gth_ref[...][:, :64], 0.0)
    z = jnp.dot(r, eg2_ref[...],
                preferred_element_type=jnp.float32) + eb2_ref[...]
    ew = jax.nn.sigmoid(z)
    o_ref[...] = jax.lax.bitcast_convert_type(ew, jnp.int32)


def _score(A, G, Eg2, eb2):
    return pl.pallas_call(
        _k_score,
        grid=(_NBLK,),
        in_specs=[pl.BlockSpec((_RB, 64), lambda i: (i, 0)),
                  pl.BlockSpec((_EB, 128), lambda i: (i, 0)),
                  pl.BlockSpec((64, 1), lambda i: (0, 0)),
                  pl.BlockSpec((1, 1), lambda i: (0, 0))],
        out_specs=pl.BlockSpec((_EB, 1), lambda i: (i, 0)),
        out_shape=jax.ShapeDtypeStruct((_E, 1), jnp.int32),
    )(A, G, Eg2.reshape(64, 1), eb2.reshape(1, 1))


_ROWS = _E // 128


def _k_mask(bits_ref, m_ref, bm_ref):
    bits = bits_ref[...]

    def cnt_ge(v):
        return jnp.sum((bits >= v).astype(jnp.int32))

    def bis_val(_, c):
        lo, hi = c
        mid = lo + (hi - lo + 1) // 2
        ok = cnt_ge(mid) >= _K
        return jnp.where(ok, mid, lo), jnp.where(ok, hi, mid - 1)

    t, _ = jax.lax.fori_loop(0, 31, bis_val,
                             (jnp.int32(0), jnp.int32(0x3F800000)))
    cnt_gt = cnt_ge(t + 1)
    extra = _K - cnt_gt
    tie = bits == t
    idx = (jax.lax.broadcasted_iota(jnp.int32, (_ROWS, 128), 0) * 128
           + jax.lax.broadcasted_iota(jnp.int32, (_ROWS, 128), 1))

    def bis_idx(_, c):
        lo, hi = c
        mid = lo + (hi - lo + 1) // 2
        ok = jnp.sum((tie & (idx <= mid)).astype(jnp.int32)) <= extra
        return jnp.where(ok, mid, lo), jnp.where(ok, hi, mid - 1)

    tau, _ = jax.lax.fori_loop(0, 19, bis_idx,
                               (jnp.int32(-1), jnp.int32(_E - 1)))
    mi = (bits > t) | (tie & (idx <= tau))
    m_ref[...] = mi.astype(jnp.float32)
    sh = jax.lax.broadcasted_iota(jnp.int32, (_ROWS // 32, 32, 128), 1)
    bm_ref[...] = jnp.sum(
        mi.astype(jnp.int32).reshape(_ROWS // 32, 32, 128) << sh, axis=1)


def _mask(bits2d):
    return pl.pallas_call(
        _k_mask,
        in_specs=[pl.BlockSpec((_ROWS, 128), lambda: (0, 0))],
        out_specs=[pl.BlockSpec((_ROWS, 128), lambda: (0, 0)),
                   pl.BlockSpec((_ROWS // 32, 128), lambda: (0, 0))],
        out_shape=[jax.ShapeDtypeStruct((_ROWS, 128), jnp.float32),
                   jax.ShapeDtypeStruct((_ROWS // 32, 128), jnp.int32)],
    )(bits2d)


def _k_final(g_ref, acc_ref, w_ref, dsq_ref, r5_ref, b1_ref, w2_ref, b2_ref,
             fw_ref, fb_ref, pool_ref, fb_out_ref, srow, s0row):
    i = pl.program_id(0)
    bat = i // 4

    @pl.when(i == 0)
    def _init():
        srow[...] = jnp.zeros_like(srow)
        s0row[...] = jnp.zeros_like(s0row)

    flag0 = (i < 4).astype(jnp.float32)
    flagn = (i >= 4).astype(jnp.float32)
    h1 = jnp.maximum(g_ref[...] * flagn + acc_ref[...] * flag0
                     + b1_ref[...], 0.0)
    wsum = jnp.dot(w_ref[...], r5_ref[...],
                   preferred_element_type=jnp.float32)
    coef = wsum + dsq_ref[...] * flag0
    part = jnp.zeros((1, _D), jnp.float32)
    for r in range(8):
        part += jnp.dot(coef[r:r + 1, :], h1[128 * r:128 * (r + 1), :],
                        preferred_element_type=jnp.float32)
    s0row[...] += part

    @pl.when(bat >= 1)
    def _colsum():
        cs = jnp.sum(h1, axis=0, keepdims=True)
        srow[pl.ds(bat, 1), :] += cs

    @pl.when(i == _NBLK - 1)
    def _fin():
        Sm = srow[...]
        Sm = jnp.concatenate([s0row[...], Sm[1:, :]], axis=0)
        pooled = jnp.dot(Sm, w2_ref[...],
                         preferred_element_type=jnp.float32) * (1.0 / _S) \
            + b2_ref[...]
        pool_ref[...] = pooled
        fb_out_ref[...] = jax.nn.sigmoid(
            jnp.dot(pooled, fw_ref[...],
                    preferred_element_type=jnp.float32) + fb_ref[...])


def _final(g, accp, w2d, dsq_nm, b1, W2, b2, Fw, Fb):
    return pl.pallas_call(
        _k_final,
        grid=(_NBLK,),
        in_specs=[pl.BlockSpec((_RB, _D), lambda i: (i, 0)),
                  pl.BlockSpec((_RB, _D), lambda i: (jnp.minimum(i, 3), 0)),
                  pl.BlockSpec((8, 640), lambda i: (i, 0)),
                  pl.BlockSpec((8, 128), lambda i: (jnp.minimum(i, 3), 0)),
                  pl.BlockSpec((640, 128), lambda i: (0, 0)),
                  pl.BlockSpec((1, _D), lambda i: (0, 0)),
                  pl.BlockSpec((_D, _D), lambda i: (0, 0)),
                  pl.BlockSpec((1, _D), lambda i: (0, 0)),
                  pl.BlockSpec((_D, _D), lambda i: (0, 0)),
                  pl.BlockSpec((1, _D), lambda i: (0, 0))],
        out_specs=[pl.BlockSpec((_B, _D), lambda i: (0, 0)),
                   pl.BlockSpec((_B, _D), lambda i: (0, 0))],
        out_shape=[jax.ShapeDtypeStruct((_B, _D), jnp.float32),
                   jax.ShapeDtypeStruct((_B, _D), jnp.float32)],
        scratch_shapes=[pltpu.VMEM((_B, _D), jnp.float32),
                        pltpu.VMEM((1, _D), jnp.float32)],
    )(g, accp, w2d, dsq_nm, jnp.asarray(_FOLD5), b1.reshape(1, _D), W2,
      b2.reshape(1, _D), Fw, Fb.reshape(1, _D))


def kernel(x, W1, b1, W2, b2, Eg1, eb1, Eg2, eb2, Fw, Fb):
    dst = jnp.asarray(_TOPO)
    xf = x.reshape(_N, _D)

    g, A = _dense(xf, W1, Eg1[:_D], eb1)
    Bm = _bmat(xf[:_S], jnp.pad(Eg1[_D:], ((0, 0), (0, 64))))
    G = _sc_gatherB(Bm, dst)
    bits = _score(A, G, Eg2, eb2)
    m2d, bm = _mask(bits.reshape(_ROWS, 128))

    degp = _sc_deg(m2d.reshape(_E), dst)
    dinv_nm, dsq_nm, mdrep_ext = _reduce_deg(degp, m2d)
    dinv0 = dinv_nm.reshape(_S)
    dinv0p = jnp.pad(dinv0, (0, 16))
    w = _sc_w(mdrep_ext.reshape(-1)[:_E], dst, dinv0p)
    accp = _sc_acc(bm.reshape(_NBM), dinv0p, g,
                   jnp.asarray(_EID2), jnp.asarray(_SRC2),
                   jnp.asarray(_DSTG2))
    return _final(g, accp, w.reshape(512, 640), dsq_nm, b1, W2, b2, Fw, Fb)
